# baseline (speedup 1.0000x reference)
import functools

import jax
import jax.numpy as jnp
from jax import lax
from jax.experimental import pallas as pl
from jax.experimental.pallas import tpu as pltpu
from jax.experimental.pallas import tpu_sc as plsc

_PADDING_IDX = 1
_NC = 2
_NS = 16
_NW = _NC * _NS
_LANES = 16
_NBUF = 4


def _bcast(x, n=_LANES):
    return lax.broadcast_in_dim(jnp.int32(x), (n,), ())


@functools.cache
def _build(seq_len: int, bsz: int, vocab: int, dim: int):
    B = seq_len * bsz
    ppw = seq_len // _NW
    n_chunks = ppw // _LANES
    assert ppw * _NW == seq_len and n_chunks * _LANES == ppw
    assert n_chunks >= _NBUF
    rpw = B // _NW
    mesh = plsc.VectorSubcoreMesh(core_axis_name="c", subcore_axis_name="s",
                                  num_cores=_NC, num_subcores=_NS)

    @functools.partial(
        pl.kernel,
        out_type=jax.ShapeDtypeStruct((B, dim), jnp.float32),
        mesh=mesh,
        compiler_params=pltpu.CompilerParams(needs_layout_passes=False),
        scratch_types=[
            pltpu.VMEM((rpw,), jnp.int32),
            pltpu.VMEM((_NBUF, _LANES, dim), jnp.float32),
            pltpu.VMEM((_LANES, dim), jnp.float32),
            pltpu.VMEM((_LANES,), jnp.int32),
            pltpu.SemaphoreType.DMA,
            pltpu.SemaphoreType.DMA,
            pltpu.SemaphoreType.DMA,
        ],
    )
    def k(tok_hbm, w_hbm, out_hbm, tok_v, wbuf, w1rep, xmin_v,
          gsem, wsem, fsem):
        wid = lax.axis_index("s") * _NC + lax.axis_index("c")
        p0 = wid * ppw
        r0 = p0 * bsz
        iota = lax.iota(jnp.int32, _LANES)

        pltpu.sync_copy(tok_hbm.at[pl.ds(r0, rpw)], tok_v)
        pltpu.async_copy(w_hbm.at[_bcast(_PADDING_IDX)], w1rep, fsem).wait()

        def gather(c, slot):
            idx = _bcast(p0 + c * _LANES + _PADDING_IDX + 1) + iota
            return pltpu.async_copy(w_hbm.at[idx], wbuf.at[slot], gsem)

        cp_g = [None] * _NBUF
        cp_w = [None] * _NBUF
        cp_g[0] = gather(0, 0)
        cp_g[1] = gather(1, 1)
        for c in range(n_chunks):
            slot = c % _NBUF
            cp_g[slot].wait()
            rbase = r0 + c * _LANES * bsz
            cp_w[slot] = []
            for b in range(bsz):
                idx = _bcast(rbase + b) + iota * bsz
                cp_w[slot].append(
                    pltpu.async_copy(wbuf.at[slot], out_hbm.at[idx], wsem))
            if c + 2 < n_chunks:
                s2 = (c + 2) % _NBUF
                if cp_w[s2] is not None:
                    for cp in cp_w[s2]:
                        cp.wait()
                cp_g[s2] = gather(c + 2, s2)
        for cps in cp_w:
            if cps is not None:
                for cp in cps:
                    cp.wait()

        for i in range(rpw // _LANES):
            t = tok_v[pl.ds(i * _LANES, _LANES)]
            m = t == _PADDING_IDX
            fv = jnp.where(m, iota, _bcast(_LANES))
            for step in (1, 2, 4, 8):
                xmin_v[...] = fv
                fv = jnp.minimum(fv, plsc.load_gather(xmin_v, [iota ^ step]))

            @pl.when(fv[0] < _LANES)
            def _():
                g = _bcast(r0 + i * _LANES) + iota
                idx = jnp.where(m, g, _bcast(r0 + i * _LANES) + fv)
                pltpu.async_copy(w1rep, out_hbm.at[idx], fsem).wait()

    return k


def kernel(input, weights):
    seq_len, bsz = input.shape
    vocab, dim = weights.shape
    k = _build(seq_len, bsz, vocab, dim)
    out = k(input.reshape(-1), weights)
    return out.reshape(seq_len, bsz, dim)


# speedup vs baseline: 2.0329x; 2.0329x over previous
import functools

import jax
import jax.numpy as jnp
from jax import lax
from jax.experimental import pallas as pl
from jax.experimental.pallas import tpu as pltpu
from jax.experimental.pallas import tpu_sc as plsc

_PADDING_IDX = 1
_NC = 2
_NS = 16
_NW = _NC * _NS
_LANES = 16
_PC = 8
_NBUF = 3


def _bcast(x, n=_LANES):
    return lax.broadcast_in_dim(jnp.int32(x), (n,), ())


@functools.cache
def _build(seq_len: int, bsz: int, vocab: int, dim: int):
    B = seq_len * bsz
    ppw = seq_len // _NW
    rpw = B // _NW
    n_chunks = ppw // _PC
    cr = _PC * bsz
    assert ppw * _NW == seq_len and n_chunks * _PC == ppw
    assert rpw % _LANES == 0 and n_chunks >= _NBUF
    mesh = plsc.VectorSubcoreMesh(core_axis_name="c", subcore_axis_name="s",
                                  num_cores=_NC, num_subcores=_NS)

    @functools.partial(
        pl.kernel,
        out_type=jax.ShapeDtypeStruct((seq_len, bsz, dim), jnp.float32),
        mesh=mesh,
        compiler_params=pltpu.CompilerParams(needs_layout_passes=False),
        scratch_types=[
            pltpu.VMEM((rpw,), jnp.int32),
            pltpu.VMEM((rpw,), jnp.int32),
            pltpu.VMEM((_NBUF, _PC, bsz, dim), jnp.float32),
            pltpu.SemaphoreType.DMA,
            pltpu.SemaphoreType.DMA,
        ],
    )
    def k(tok_hbm, w_hbm, out_hbm, tok_v, idx_v, wbuf, gsem, wsem):
        wid = lax.axis_index("s") * _NC + lax.axis_index("c")
        p0 = wid * ppw
        r0 = p0 * bsz
        iota = lax.iota(jnp.int32, _LANES)

        pltpu.sync_copy(tok_hbm.at[pl.ds(r0, rpw)], tok_v)
        for i in range(rpw // _LANES):
            t = tok_v[pl.ds(i * _LANES, _LANES)]
            g = _bcast(r0 + i * _LANES) + iota
            gpos = (g >> bsz.bit_length() - 1 if bsz & (bsz - 1) == 0
                    else g // bsz) + (_PADDING_IDX + 1)
            idx_v[pl.ds(i * _LANES, _LANES)] = jnp.where(t != _PADDING_IDX,
                                                         gpos, t)

        def gather(c, slot):
            return pltpu.async_copy(
                w_hbm.at[idx_v.at[pl.ds(c * cr, cr)]],
                wbuf.at[slot].reshape(cr, dim), gsem)

        cp_g = [None] * _NBUF
        cp_w = [None] * _NBUF
        cp_g[0] = gather(0, 0)
        cp_g[1] = gather(1, 1)
        for c in range(n_chunks):
            slot = c % _NBUF
            cp_g[slot].wait()
            cp_w[slot] = pltpu.async_copy(
                wbuf.at[slot], out_hbm.at[pl.ds(p0 + c * _PC, _PC)], wsem)
            if c + 2 < n_chunks:
                s2 = (c + 2) % _NBUF
                if cp_w[s2] is not None:
                    cp_w[s2].wait()
                cp_g[s2] = gather(c + 2, s2)
        for cp in cp_w:
            if cp is not None:
                cp.wait()

    return k


def kernel(input, weights):
    seq_len, bsz = input.shape
    vocab, dim = weights.shape
    k = _build(seq_len, bsz, vocab, dim)
    return k(input.reshape(-1), weights)
